# baseline (speedup 1.0000x reference)
import jax
import jax.numpy as jnp
from jax import lax
from jax.experimental import pallas as pl
from jax.experimental.pallas import tpu as pltpu
from jax.experimental.pallas import tpu_sc as plsc

_DET_THRESH = 0.5


def _dense_kernel(obj_ref, oi_ref, pxy_ref, gtt_ref, gidrow_ref, pxyt_ref, gtb_ref,
                  dist_ref, distt_ref, score_ref, gtidx0_ref, objix0_ref,
                  dbg0_ref, colmin_ref, colarg_ref, rowpen_ref):
    P = obj_ref.shape[0]
    M = gtt_ref.shape[1]
    PPAD = pxyt_ref.shape[1]
    INF = jnp.float32(jnp.inf)
    i32 = jnp.int32
    BIG = jnp.int32(2**30)

    x = pxy_ref[:, 0:1]
    y = pxy_ref[:, 1:2]
    gx = gtt_ref[0:1, :]
    gy = gtt_ref[1:2, :]
    dist = (x - gx) ** 2 + (y - gy) ** 2
    dist_ref[...] = dist

    xt = pxyt_ref[0:1, :]
    yt = pxyt_ref[1:2, :]
    gxc = gtb_ref[:, 0:1]
    gyc = gtb_ref[:, 1:2]
    lane_iota = lax.broadcasted_iota(i32, (M, PPAD), 1)
    distt_ref[...] = jnp.where(lane_iota < P,
                               (gxc - xt) ** 2 + (gyc - yt) ** 2, INF)

    eq = oi_ref[...] == gidrow_ref[...]
    j_iota = lax.broadcasted_iota(i32, (P, M), 1)
    firstj = jnp.min(jnp.where(eq, j_iota, M), axis=1, keepdims=True)
    has_pr = firstj < M
    a_gt0 = jnp.max(eq.astype(i32), axis=0, keepdims=True) > 0

    rowpen = jnp.where(has_pr, INF, jnp.float32(0.0))
    rowpen_ref[...] = rowpen

    i_iota = lax.broadcasted_iota(i32, (P, M), 0)
    d0 = dist + rowpen
    cm0 = jnp.min(d0, axis=0, keepdims=True)
    ca0 = jnp.min(jnp.where(d0 == cm0, i_iota, BIG), axis=0, keepdims=True)
    colmin_ref[...] = jnp.where(a_gt0, INF, cm0)
    colarg_ref[...] = ca0.astype(i32)

    gtidx0_ref[...] = jnp.where(has_pr, firstj, jnp.int32(-1)).astype(i32)
    objix0_ref[...] = oi_ref[...]
    score_ref[...] = jax.nn.sigmoid(obj_ref[...])
    dbg0_ref[...] = (jnp.where(has_pr, jnp.int32(2), jnp.int32(0))
                     + jnp.where(obj_ref[...] > _DET_THRESH,
                                 jnp.int32(10), jnp.int32(0))).astype(i32)


def _sc_greedy(cm_hbm, ca_hbm, rp_hbm, distt_hbm, gid_hbm,
               gt0_hbm, ob0_hbm, db0_hbm,
               gt_out, ob_out, db_out,
               cm_v, ca_v, rp_v, gid_v, gt_v, ob_v, db_v, buf_v):
    M = 64
    NB = M // 16
    PPAD = rp_hbm.shape[0]
    PC = PPAD // 16
    INF = jnp.float32(jnp.inf)
    i32 = jnp.int32
    BIG = jnp.int32(2**30)

    @pl.when((lax.axis_index("c") == 0) & (lax.axis_index("s") == 0))
    def _():
        lane = lax.broadcasted_iota(i32, (16,), 0)
        lane0 = lane == 0

        pltpu.sync_copy(cm_hbm, cm_v)
        pltpu.sync_copy(ca_hbm, ca_v)
        pltpu.sync_copy(gid_hbm, gid_v)
        pltpu.sync_copy(rp_hbm, rp_v)
        pltpu.sync_copy(gt0_hbm, gt_v)
        pltpu.sync_copy(ob0_hbm, ob_v)
        pltpu.sync_copy(db0_hbm, db_v)

        def round_body(r, carry):
            cms = [cm_v[pl.ds(16 * b, 16)] for b in range(NB)]
            cas = [ca_v[pl.ds(16 * b, 16)] for b in range(NB)]
            m = jnp.min(cms[0])
            for b in range(1, NB):
                m = jnp.minimum(m, jnp.min(cms[b]))
            ks = [jnp.min(jnp.where(cms[b] == m,
                                    cas[b] * M + (lane + 16 * b), BIG))
                  for b in range(NB)]
            k = ks[0]
            for b in range(1, NB):
                k = jnp.minimum(k, ks[b])
            i = k // M
            j = k - i * M

            @pl.when(m < INF)
            def _():
                iv = jnp.full((16,), i, i32)
                jv = jnp.full((16,), j, i32)
                plsc.store_scatter(gt_v, [iv], jv, mask=lane0)
                gidj = plsc.load_gather(gid_v, [jv])
                plsc.store_scatter(ob_v, [iv], gidj, mask=lane0)
                dbi = plsc.load_gather(db_v, [iv])
                plsc.store_scatter(db_v, [iv], dbi + 3, mask=lane0)
                plsc.store_scatter(rp_v, [iv], jnp.full((16,), INF), mask=lane0)
                n_stale = jnp.int32(0)
                for b in range(NB):
                    cmb = jnp.where(lane + 16 * b == j, INF, cms[b])
                    cm_v[pl.ds(16 * b, 16)] = cmb
                    n_stale = n_stale + jnp.max(
                        jnp.where((cas[b] == i) & (cmb < INF),
                                  jnp.int32(1), jnp.int32(0)))

                @pl.when(n_stale > 0)
                def _():
                    def rescan_col(j2, c2):
                        j2v = jnp.full((16,), j2, i32)
                        cmj = jnp.min(plsc.load_gather(cm_v, [j2v]))
                        caj = jnp.min(plsc.load_gather(ca_v, [j2v]))

                        @pl.when((caj == i) & (cmj < INF))
                        def _():
                            off = pl.multiple_of(j2 * PPAD, 1024)
                            pltpu.sync_copy(distt_hbm.at[pl.ds(off, PPAD)],
                                            buf_v)

                            def scan_chunk(c, mv_av):
                                mv, av = mv_av
                                idx = lane + c * 16
                                v = (plsc.load_gather(buf_v, [idx])
                                     + plsc.load_gather(rp_v, [idx]))
                                lt = v < mv
                                av2 = jnp.where(lt, idx, av)
                                mv2 = jnp.where(lt, v, mv)
                                return (mv2, av2)

                            mv, av = lax.fori_loop(
                                jnp.int32(0), jnp.int32(PC), scan_chunk,
                                (jnp.full((16,), INF), jnp.full((16,), BIG, i32)))
                            mcol = jnp.min(mv)
                            acol = jnp.min(jnp.where(mv == mcol, av, BIG))
                            plsc.store_scatter(
                                cm_v, [j2v], jnp.full((16,), mcol), mask=lane0)
                            plsc.store_scatter(
                                ca_v, [j2v], jnp.full((16,), acol, i32),
                                mask=lane0)
                        return c2

                    lax.fori_loop(jnp.int32(0), jnp.int32(M), rescan_col,
                                  jnp.int32(0))

            return carry

        lax.fori_loop(jnp.int32(0), jnp.int32(M), round_body, jnp.int32(0))

        pltpu.sync_copy(gt_v, gt_out)
        pltpu.sync_copy(ob_v, ob_out)
        pltpu.sync_copy(db_v, db_out)


def kernel(is_object, position, output_hs, gt_boxes, obj_idx, gt_obj_ids):
    P = obj_idx.shape[0]
    M = gt_obj_ids.shape[0]
    PPAD = -(-P // 1024) * 1024
    MPAD = 1024
    INF = jnp.float32(jnp.inf)

    obj = is_object[-1, 0, :, :]
    oi = obj_idx.astype(jnp.int32).reshape(P, 1)
    pxy = position[-1, 0, :, :2]
    pxyt = jnp.pad(pxy.T, ((0, 0), (0, PPAD - P)))
    gtb = gt_boxes[:, :2]
    gtt = gtb.T
    gid32 = gt_obj_ids.astype(jnp.int32)
    gidrow = gid32.reshape(1, M)

    (dist, distt, score, gtidx0, objix0, dbg0,
     colmin0, colarg0, rowpen0) = pl.pallas_call(
        _dense_kernel,
        out_shape=[
            jax.ShapeDtypeStruct((P, M), jnp.float32),
            jax.ShapeDtypeStruct((M, PPAD), jnp.float32),
            jax.ShapeDtypeStruct((P, 1), jnp.float32),
            jax.ShapeDtypeStruct((P, 1), jnp.int32),
            jax.ShapeDtypeStruct((P, 1), jnp.int32),
            jax.ShapeDtypeStruct((P, 1), jnp.int32),
            jax.ShapeDtypeStruct((1, M), jnp.float32),
            jax.ShapeDtypeStruct((1, M), jnp.int32),
            jax.ShapeDtypeStruct((P, 1), jnp.float32),
        ],
    )(obj, oi, pxy, gtt, gidrow, pxyt, gtb)

    cm_p = jnp.pad(colmin0.reshape(M), (0, MPAD - M), constant_values=jnp.inf)
    ca_p = jnp.pad(colarg0.reshape(M), (0, MPAD - M), constant_values=2**30)
    gid_p = jnp.pad(gid32, (0, MPAD - M))
    rp_p = jnp.pad(rowpen0.reshape(P), (0, PPAD - P), constant_values=jnp.inf)
    gt0_p = jnp.pad(gtidx0.reshape(P), (0, PPAD - P))
    ob0_p = jnp.pad(objix0.reshape(P), (0, PPAD - P))
    db0_p = jnp.pad(dbg0.reshape(P), (0, PPAD - P))
    distt_flat = distt.reshape(M * PPAD)

    mesh = plsc.VectorSubcoreMesh(core_axis_name="c", subcore_axis_name="s")
    sc_fn = pl.kernel(
        _sc_greedy,
        out_type=[
            jax.ShapeDtypeStruct((PPAD,), jnp.int32),
            jax.ShapeDtypeStruct((PPAD,), jnp.int32),
            jax.ShapeDtypeStruct((PPAD,), jnp.int32),
        ],
        mesh=mesh,
        compiler_params=pltpu.CompilerParams(needs_layout_passes=False),
        scratch_types=[
            pltpu.VMEM((MPAD,), jnp.float32),
            pltpu.VMEM((MPAD,), jnp.int32),
            pltpu.VMEM((PPAD,), jnp.float32),
            pltpu.VMEM((MPAD,), jnp.int32),
            pltpu.VMEM((PPAD,), jnp.int32),
            pltpu.VMEM((PPAD,), jnp.int32),
            pltpu.VMEM((PPAD,), jnp.int32),
            pltpu.VMEM((PPAD,), jnp.float32),
        ],
    )
    gtidx, objix, dbg = sc_fn(cm_p, ca_p, rp_p, distt_flat, gid_p,
                              gt0_p, ob0_p, db0_p)

    score = score.reshape(P)
    gt_idx = gtidx[:P].astype(jnp.int64)
    obj_ix = objix[:P].astype(jnp.int64)
    dbg = dbg[:P]
    q_ref = position[-1, 0]
    q_emb = output_hs[-1, 0]
    return (score, dist, dbg, gt_idx, obj_ix, q_ref, q_emb)


# speedup vs baseline: 1.0533x; 1.0533x over previous
import jax
import jax.numpy as jnp
from jax import lax
from jax.experimental import pallas as pl
from jax.experimental.pallas import tpu as pltpu
from jax.experimental.pallas import tpu_sc as plsc

_DET_THRESH = 0.5


def _dense_kernel(obj_ref, oi_ref, pxy_ref, gtt_ref, gidrow_ref, pxyt_ref, gtb_ref,
                  dist_ref, distt_ref, score_ref, gtidx0_ref, objix0_ref,
                  dbg0_ref, colmin_ref, colarg_ref, rowpen_ref):
    P = obj_ref.shape[0]
    M = gtt_ref.shape[1]
    PPAD = pxyt_ref.shape[1]
    INF = jnp.float32(jnp.inf)
    i32 = jnp.int32
    BIG = jnp.int32(2**30)

    x = pxy_ref[:, 0:1]
    y = pxy_ref[:, 1:2]
    gx = gtt_ref[0:1, :]
    gy = gtt_ref[1:2, :]
    dist = (x - gx) ** 2 + (y - gy) ** 2
    dist_ref[...] = dist

    xt = pxyt_ref[0:1, :]
    yt = pxyt_ref[1:2, :]
    gxc = gtb_ref[:, 0:1]
    gyc = gtb_ref[:, 1:2]
    lane_iota = lax.broadcasted_iota(i32, (M, PPAD), 1)
    distt_ref[...] = jnp.where(lane_iota < P,
                               (gxc - xt) ** 2 + (gyc - yt) ** 2, INF)

    eq = oi_ref[...] == gidrow_ref[...]
    j_iota = lax.broadcasted_iota(i32, (P, M), 1)
    firstj = jnp.min(jnp.where(eq, j_iota, M), axis=1, keepdims=True)
    has_pr = firstj < M
    a_gt0 = jnp.max(eq.astype(i32), axis=0, keepdims=True) > 0

    rowpen = jnp.where(has_pr, INF, jnp.float32(0.0))
    rowpen_ref[...] = rowpen

    i_iota = lax.broadcasted_iota(i32, (P, M), 0)
    d0 = dist + rowpen
    cm0 = jnp.min(d0, axis=0, keepdims=True)
    ca0 = jnp.min(jnp.where(d0 == cm0, i_iota, BIG), axis=0, keepdims=True)
    colmin_ref[...] = jnp.where(a_gt0, INF, cm0)
    colarg_ref[...] = ca0.astype(i32)

    gtidx0_ref[...] = jnp.where(has_pr, firstj, jnp.int32(-1)).astype(i32)
    objix0_ref[...] = oi_ref[...]
    score_ref[...] = jax.nn.sigmoid(obj_ref[...])
    dbg0_ref[...] = (jnp.where(has_pr, jnp.int32(2), jnp.int32(0))
                     + jnp.where(obj_ref[...] > _DET_THRESH,
                                 jnp.int32(10), jnp.int32(0))).astype(i32)


def _sc_greedy(cm_hbm, ca_hbm, rp_hbm, distt_hbm, gid_hbm,
               gt0_hbm, ob0_hbm, db0_hbm,
               gt_out, ob_out, db_out,
               cm_v, ca_v, rp_v, gid_v, gt_v, ob_v, db_v, buf_v, dma_sem):
    M = 64
    NB = M // 16
    PPAD = rp_hbm.shape[0]
    PC = PPAD // 16
    INF = jnp.float32(jnp.inf)
    i32 = jnp.int32
    BIG = jnp.int32(2**30)

    @pl.when((lax.axis_index("c") == 0) & (lax.axis_index("s") == 0))
    def _():
        lane = lax.broadcasted_iota(i32, (16,), 0)
        lane0 = lane == 0

        copies = [pltpu.async_copy(s, d, dma_sem)
                  for s, d in ((cm_hbm, cm_v), (ca_hbm, ca_v),
                               (gid_hbm, gid_v), (rp_hbm, rp_v),
                               (gt0_hbm, gt_v), (ob0_hbm, ob_v),
                               (db0_hbm, db_v))]
        for c in copies:
            c.wait()

        def round_body(r, carry):
            cms = [cm_v[pl.ds(16 * b, 16)] for b in range(NB)]
            cas = [ca_v[pl.ds(16 * b, 16)] for b in range(NB)]
            mv = cms[0]
            for b in range(1, NB):
                mv = jnp.minimum(mv, cms[b])
            m = jnp.min(mv)
            kv = jnp.where(cms[0] == m, cas[0] * M + lane, BIG)
            for b in range(1, NB):
                kv = jnp.minimum(
                    kv, jnp.where(cms[b] == m,
                                  cas[b] * M + (lane + 16 * b), BIG))
            k = jnp.min(kv)
            i = k // M
            j = k - i * M

            @pl.when(m < INF)
            def _():
                iv = jnp.full((16,), i, i32)
                jv = jnp.full((16,), j, i32)
                plsc.store_scatter(gt_v, [iv], jv, mask=lane0)
                gidj = plsc.load_gather(gid_v, [jv])
                plsc.store_scatter(ob_v, [iv], gidj, mask=lane0)
                dbi = plsc.load_gather(db_v, [iv])
                plsc.store_scatter(db_v, [iv], dbi + 3, mask=lane0)
                plsc.store_scatter(rp_v, [iv], jnp.full((16,), INF), mask=lane0)
                stale_v = jnp.zeros((16,), i32)
                for b in range(NB):
                    cmb = jnp.where(lane + 16 * b == j, INF, cms[b])
                    cm_v[pl.ds(16 * b, 16)] = cmb
                    stale_v = stale_v + jnp.where(
                        (cas[b] == i) & (cmb < INF),
                        jnp.int32(1), jnp.int32(0))
                n_stale = jnp.max(stale_v)

                @pl.when(n_stale > 0)
                def _():
                    def rescan_col(j2, c2):
                        j2v = jnp.full((16,), j2, i32)
                        cmj = jnp.min(plsc.load_gather(cm_v, [j2v]))
                        caj = jnp.min(plsc.load_gather(ca_v, [j2v]))

                        @pl.when((caj == i) & (cmj < INF))
                        def _():
                            off = pl.multiple_of(j2 * PPAD, 1024)
                            pltpu.sync_copy(distt_hbm.at[pl.ds(off, PPAD)],
                                            buf_v)

                            def scan_chunk(c, mv_av):
                                mv, av = mv_av
                                idx = lane + c * 16
                                v = (plsc.load_gather(buf_v, [idx])
                                     + plsc.load_gather(rp_v, [idx]))
                                lt = v < mv
                                av2 = jnp.where(lt, idx, av)
                                mv2 = jnp.where(lt, v, mv)
                                return (mv2, av2)

                            mv, av = lax.fori_loop(
                                jnp.int32(0), jnp.int32(PC), scan_chunk,
                                (jnp.full((16,), INF), jnp.full((16,), BIG, i32)))
                            mcol = jnp.min(mv)
                            acol = jnp.min(jnp.where(mv == mcol, av, BIG))
                            plsc.store_scatter(
                                cm_v, [j2v], jnp.full((16,), mcol), mask=lane0)
                            plsc.store_scatter(
                                ca_v, [j2v], jnp.full((16,), acol, i32),
                                mask=lane0)
                        return c2

                    lax.fori_loop(jnp.int32(0), jnp.int32(M), rescan_col,
                                  jnp.int32(0))

            return carry

        lax.fori_loop(jnp.int32(0), jnp.int32(M), round_body, jnp.int32(0))

        out_copies = [pltpu.async_copy(s, d, dma_sem)
                      for s, d in ((gt_v, gt_out), (ob_v, ob_out),
                                   (db_v, db_out))]
        for c in out_copies:
            c.wait()


def kernel(is_object, position, output_hs, gt_boxes, obj_idx, gt_obj_ids):
    P = obj_idx.shape[0]
    M = gt_obj_ids.shape[0]
    PPAD = -(-P // 1024) * 1024
    MPAD = 1024
    INF = jnp.float32(jnp.inf)

    obj = is_object[-1, 0, :, :]
    oi = obj_idx.astype(jnp.int32).reshape(P, 1)
    pxy = position[-1, 0, :, :2]
    pxyt = jnp.pad(pxy.T, ((0, 0), (0, PPAD - P)))
    gtb = gt_boxes[:, :2]
    gtt = gtb.T
    gid32 = gt_obj_ids.astype(jnp.int32)
    gidrow = gid32.reshape(1, M)

    (dist, distt, score, gtidx0, objix0, dbg0,
     colmin0, colarg0, rowpen0) = pl.pallas_call(
        _dense_kernel,
        out_shape=[
            jax.ShapeDtypeStruct((P, M), jnp.float32),
            jax.ShapeDtypeStruct((M, PPAD), jnp.float32),
            jax.ShapeDtypeStruct((P, 1), jnp.float32),
            jax.ShapeDtypeStruct((P, 1), jnp.int32),
            jax.ShapeDtypeStruct((P, 1), jnp.int32),
            jax.ShapeDtypeStruct((P, 1), jnp.int32),
            jax.ShapeDtypeStruct((1, M), jnp.float32),
            jax.ShapeDtypeStruct((1, M), jnp.int32),
            jax.ShapeDtypeStruct((P, 1), jnp.float32),
        ],
    )(obj, oi, pxy, gtt, gidrow, pxyt, gtb)

    cm_p = jnp.pad(colmin0.reshape(M), (0, MPAD - M), constant_values=jnp.inf)
    ca_p = jnp.pad(colarg0.reshape(M), (0, MPAD - M), constant_values=2**30)
    gid_p = jnp.pad(gid32, (0, MPAD - M))
    rp_p = jnp.pad(rowpen0.reshape(P), (0, PPAD - P), constant_values=jnp.inf)
    gt0_p = jnp.pad(gtidx0.reshape(P), (0, PPAD - P))
    ob0_p = jnp.pad(objix0.reshape(P), (0, PPAD - P))
    db0_p = jnp.pad(dbg0.reshape(P), (0, PPAD - P))
    distt_flat = distt.reshape(M * PPAD)

    mesh = plsc.VectorSubcoreMesh(core_axis_name="c", subcore_axis_name="s")
    sc_fn = pl.kernel(
        _sc_greedy,
        out_type=[
            jax.ShapeDtypeStruct((PPAD,), jnp.int32),
            jax.ShapeDtypeStruct((PPAD,), jnp.int32),
            jax.ShapeDtypeStruct((PPAD,), jnp.int32),
        ],
        mesh=mesh,
        compiler_params=pltpu.CompilerParams(needs_layout_passes=False),
        scratch_types=[
            pltpu.VMEM((MPAD,), jnp.float32),
            pltpu.VMEM((MPAD,), jnp.int32),
            pltpu.VMEM((PPAD,), jnp.float32),
            pltpu.VMEM((MPAD,), jnp.int32),
            pltpu.VMEM((PPAD,), jnp.int32),
            pltpu.VMEM((PPAD,), jnp.int32),
            pltpu.VMEM((PPAD,), jnp.int32),
            pltpu.VMEM((PPAD,), jnp.float32),
            pltpu.SemaphoreType.DMA,
        ],
    )
    gtidx, objix, dbg = sc_fn(cm_p, ca_p, rp_p, distt_flat, gid_p,
                              gt0_p, ob0_p, db0_p)

    score = score.reshape(P)
    gt_idx = gtidx[:P].astype(jnp.int64)
    obj_ix = objix[:P].astype(jnp.int64)
    dbg = dbg[:P]
    q_ref = position[-1, 0]
    q_emb = output_hs[-1, 0]
    return (score, dist, dbg, gt_idx, obj_ix, q_ref, q_emb)


# speedup vs baseline: 1.4694x; 1.3950x over previous
import jax
import jax.numpy as jnp
from jax import lax
from jax.experimental import pallas as pl
from jax.experimental.pallas import tpu as pltpu
from jax.experimental.pallas import tpu_sc as plsc

_DET_THRESH = 0.5


def _dense_kernel(obj_ref, oi_ref, pxy_ref, gtt_ref, gidcol_ref, pxyt_ref, gtb_ref,
                  dist_ref, distt_ref, score_ref, gt0_ref, ob0_ref,
                  db0_ref, cm_ref, ca_ref, rp_ref):
    P = pxy_ref.shape[0]
    M = gtt_ref.shape[1]
    PPAD = pxyt_ref.shape[1]
    MPAD = cm_ref.shape[0]
    INF = jnp.float32(jnp.inf)
    i32 = jnp.int32
    BIG = jnp.int32(2**30)

    x = pxy_ref[:, 0:1]
    y = pxy_ref[:, 1:2]
    gx = gtt_ref[0:1, :]
    gy = gtt_ref[1:2, :]
    dist_ref[...] = (x - gx) ** 2 + (y - gy) ** 2

    xt = pxyt_ref[0:1, :]
    yt = pxyt_ref[1:2, :]
    gxc = gtb_ref[:, 0:1]
    gyc = gtb_ref[:, 1:2]
    lane_mp = lax.broadcasted_iota(i32, (M, PPAD), 1)
    dist_t = jnp.where(lane_mp < P, (gxc - xt) ** 2 + (gyc - yt) ** 2, INF)
    distt_ref[...] = dist_t

    eq_t = (gidcol_ref[...] == oi_ref[...]) & (lane_mp < P)
    j_iota = lax.broadcasted_iota(i32, (M, PPAD), 0)
    firstj = jnp.min(jnp.where(eq_t, j_iota, M), axis=0, keepdims=True)
    has_pr = firstj < M
    a_gt0 = jnp.max(eq_t.astype(i32), axis=1, keepdims=True) > 0

    lane_row = lax.broadcasted_iota(i32, (1, PPAD), 1)
    rowpen = jnp.where(has_pr | (lane_row >= P), INF, jnp.float32(0.0))
    rp_ref[...] = rowpen.reshape(PPAD)

    d0 = dist_t + rowpen
    cmraw = jnp.min(d0, axis=1, keepdims=True)
    ca_col = jnp.min(jnp.where(d0 == cmraw, lane_mp, BIG),
                     axis=1, keepdims=True)
    cm_col = jnp.where(a_gt0, INF, cmraw)
    cm_row = jnp.transpose(cm_col)
    ca_row = jnp.transpose(ca_col).astype(i32)
    cm_ref[pl.ds(0, M)] = cm_row.reshape(M)
    cm_ref[pl.ds(M, MPAD - M)] = jnp.full((MPAD - M,), INF)
    ca_ref[pl.ds(0, M)] = ca_row.reshape(M)
    ca_ref[pl.ds(M, MPAD - M)] = jnp.full((MPAD - M,), BIG, i32)

    gt0_ref[...] = jnp.where(has_pr, firstj,
                             jnp.int32(-1)).astype(i32).reshape(PPAD)
    ob0_ref[...] = jnp.where(lane_row < P, oi_ref[...],
                             jnp.int32(0)).astype(i32).reshape(PPAD)
    db0_ref[...] = (jnp.where(has_pr, jnp.int32(2), jnp.int32(0))
                    + jnp.where(obj_ref[...] > _DET_THRESH,
                                jnp.int32(10), jnp.int32(0))).reshape(PPAD)
    score_ref[...] = jax.nn.sigmoid(obj_ref[:, 0:P]).reshape(P)


def _sc_greedy(cm_hbm, ca_hbm, rp_hbm, distt_hbm, gid_hbm,
               gt0_hbm, ob0_hbm, db0_hbm,
               gt_out, ob_out, db_out,
               cm_v, ca_v, rp_v, gid_v, gt_v, ob_v, db_v, buf_v, dma_sem):
    M = 64
    NB = M // 16
    PPAD = rp_hbm.shape[0]
    PC = PPAD // 16
    INF = jnp.float32(jnp.inf)
    i32 = jnp.int32
    BIG = jnp.int32(2**30)

    @pl.when((lax.axis_index("c") == 0) & (lax.axis_index("s") == 0))
    def _():
        lane = lax.broadcasted_iota(i32, (16,), 0)
        lane0 = lane == 0

        copies = [pltpu.async_copy(s, d, dma_sem)
                  for s, d in ((cm_hbm, cm_v), (ca_hbm, ca_v),
                               (gid_hbm, gid_v), (rp_hbm, rp_v),
                               (gt0_hbm, gt_v), (ob0_hbm, ob_v),
                               (db0_hbm, db_v))]
        for c in copies:
            c.wait()

        def round_body(r, carry):
            cms = [cm_v[pl.ds(16 * b, 16)] for b in range(NB)]
            cas = [ca_v[pl.ds(16 * b, 16)] for b in range(NB)]
            mv = cms[0]
            for b in range(1, NB):
                mv = jnp.minimum(mv, cms[b])
            m = jnp.min(mv)
            kv = jnp.where(cms[0] == m, cas[0] * M + lane, BIG)
            for b in range(1, NB):
                kv = jnp.minimum(
                    kv, jnp.where(cms[b] == m,
                                  cas[b] * M + (lane + 16 * b), BIG))
            k = jnp.min(kv)
            i = k // M
            j = k - i * M

            @pl.when(m < INF)
            def _():
                iv = jnp.full((16,), i, i32)
                jv = jnp.full((16,), j, i32)
                plsc.store_scatter(gt_v, [iv], jv, mask=lane0)
                gidj = plsc.load_gather(gid_v, [jv])
                plsc.store_scatter(ob_v, [iv], gidj, mask=lane0)
                dbi = plsc.load_gather(db_v, [iv])
                plsc.store_scatter(db_v, [iv], dbi + 3, mask=lane0)
                plsc.store_scatter(rp_v, [iv], jnp.full((16,), INF), mask=lane0)
                stale_v = jnp.zeros((16,), i32)
                for b in range(NB):
                    cmb = jnp.where(lane + 16 * b == j, INF, cms[b])
                    cm_v[pl.ds(16 * b, 16)] = cmb
                    stale_v = stale_v + jnp.where(
                        (cas[b] == i) & (cmb < INF),
                        jnp.int32(1), jnp.int32(0))
                n_stale = jnp.max(stale_v)

                @pl.when(n_stale > 0)
                def _():
                    def rescan_col(j2, c2):
                        j2v = jnp.full((16,), j2, i32)
                        cmj = jnp.min(plsc.load_gather(cm_v, [j2v]))
                        caj = jnp.min(plsc.load_gather(ca_v, [j2v]))

                        @pl.when((caj == i) & (cmj < INF))
                        def _():
                            off = pl.multiple_of(j2 * PPAD, 1024)
                            pltpu.sync_copy(distt_hbm.at[pl.ds(off, PPAD)],
                                            buf_v)

                            def scan_chunk(c, mv_av):
                                mv2, av2 = mv_av
                                idx = lane + c * 16
                                v = (plsc.load_gather(buf_v, [idx])
                                     + plsc.load_gather(rp_v, [idx]))
                                lt = v < mv2
                                av3 = jnp.where(lt, idx, av2)
                                mv3 = jnp.where(lt, v, mv2)
                                return (mv3, av3)

                            mvf, avf = lax.fori_loop(
                                jnp.int32(0), jnp.int32(PC), scan_chunk,
                                (jnp.full((16,), INF), jnp.full((16,), BIG, i32)))
                            mcol = jnp.min(mvf)
                            acol = jnp.min(jnp.where(mvf == mcol, avf, BIG))
                            plsc.store_scatter(
                                cm_v, [j2v], jnp.full((16,), mcol), mask=lane0)
                            plsc.store_scatter(
                                ca_v, [j2v], jnp.full((16,), acol, i32),
                                mask=lane0)
                        return c2

                    lax.fori_loop(jnp.int32(0), jnp.int32(M), rescan_col,
                                  jnp.int32(0))

            return carry

        lax.fori_loop(jnp.int32(0), jnp.int32(M), round_body, jnp.int32(0))

        out_copies = [pltpu.async_copy(s, d, dma_sem)
                      for s, d in ((gt_v, gt_out), (ob_v, ob_out),
                                   (db_v, db_out))]
        for c in out_copies:
            c.wait()


def kernel(is_object, position, output_hs, gt_boxes, obj_idx, gt_obj_ids):
    P = obj_idx.shape[0]
    M = gt_obj_ids.shape[0]
    PPAD = -(-P // 1024) * 1024
    MPAD = 1024

    obj = jnp.pad(is_object[-1, 0, :, 0].reshape(1, P),
                  ((0, 0), (0, PPAD - P)))
    oi = jnp.pad(obj_idx.astype(jnp.int32).reshape(1, P),
                 ((0, 0), (0, PPAD - P)), constant_values=-1)
    pxy = position[-1, 0, :, :2]
    pxyt = jnp.pad(pxy.T, ((0, 0), (0, PPAD - P)))
    gtb = gt_boxes[:, :2]
    gtt = gtb.T
    gid32 = gt_obj_ids.astype(jnp.int32)
    gidcol = gid32.reshape(M, 1)
    gid_p = jnp.pad(gid32, (0, MPAD - M))

    (dist, distt, score, gt0_p, ob0_p, db0_p,
     cm_p, ca_p, rp_p) = pl.pallas_call(
        _dense_kernel,
        out_shape=[
            jax.ShapeDtypeStruct((P, M), jnp.float32),
            jax.ShapeDtypeStruct((M, PPAD), jnp.float32),
            jax.ShapeDtypeStruct((P,), jnp.float32),
            jax.ShapeDtypeStruct((PPAD,), jnp.int32),
            jax.ShapeDtypeStruct((PPAD,), jnp.int32),
            jax.ShapeDtypeStruct((PPAD,), jnp.int32),
            jax.ShapeDtypeStruct((MPAD,), jnp.float32),
            jax.ShapeDtypeStruct((MPAD,), jnp.int32),
            jax.ShapeDtypeStruct((PPAD,), jnp.float32),
        ],
    )(obj, oi, pxy, gtt, gidcol, pxyt, gtb)

    distt_flat = distt.reshape(M * PPAD)

    mesh = plsc.VectorSubcoreMesh(core_axis_name="c", subcore_axis_name="s")
    sc_fn = pl.kernel(
        _sc_greedy,
        out_type=[
            jax.ShapeDtypeStruct((PPAD,), jnp.int32),
            jax.ShapeDtypeStruct((PPAD,), jnp.int32),
            jax.ShapeDtypeStruct((PPAD,), jnp.int32),
        ],
        mesh=mesh,
        compiler_params=pltpu.CompilerParams(needs_layout_passes=False),
        scratch_types=[
            pltpu.VMEM((MPAD,), jnp.float32),
            pltpu.VMEM((MPAD,), jnp.int32),
            pltpu.VMEM((PPAD,), jnp.float32),
            pltpu.VMEM((MPAD,), jnp.int32),
            pltpu.VMEM((PPAD,), jnp.int32),
            pltpu.VMEM((PPAD,), jnp.int32),
            pltpu.VMEM((PPAD,), jnp.int32),
            pltpu.VMEM((PPAD,), jnp.float32),
            pltpu.SemaphoreType.DMA,
        ],
    )
    gtidx, objix, dbg = sc_fn(cm_p, ca_p, rp_p, distt_flat, gid_p,
                              gt0_p, ob0_p, db0_p)

    gt_idx = gtidx[:P].astype(jnp.int64)
    obj_ix = objix[:P].astype(jnp.int64)
    dbg = dbg[:P]
    q_ref = position[-1, 0]
    q_emb = output_hs[-1, 0]
    return (score, dist, dbg, gt_idx, obj_ix, q_ref, q_emb)
